# baseline (speedup 1.0000x reference)
import functools

import jax
import jax.numpy as jnp
from jax import lax
from jax.experimental import pallas as pl
from jax.experimental.pallas import tpu as pltpu
from jax.experimental.pallas import tpu_sc as plsc

B = 16384
E = 16
ROWS = 128
CHUNK = 128
RPT = 8
RPW = 4
NVEC = CHUNK // 16


def _body(uflat, mflat, ubias, mbias, uidx, midx, out,
          uidx_v, midx_v, flatidx, uvals, mvals,
          accv, shared, sumv, ubv, mbv, outv, sem):
    c = lax.axis_index("c")
    s = lax.axis_index("s")

    pltpu.sync_copy(uidx.at[pl.ds(s * RPT, RPT)], uidx_v)
    pltpu.sync_copy(midx.at[pl.ds(s * RPT, RPT)], midx_v)

    def dim_step(e, acc):
        base = e * 1_000_000
        for j in range(RPT):
            for v in range(NVEC):
                sl = pl.ds(v * 16, 16)
                flatidx[0, j, sl] = uidx_v[j, sl] + base
                flatidx[1, j, sl] = midx_v[j, sl] + base
        cps = []
        for j in range(RPT):
            cps.append(pltpu.async_copy(
                uflat.at[flatidx.at[0, j]], uvals.at[j], sem))
            cps.append(pltpu.async_copy(
                mflat.at[flatidx.at[1, j]], mvals.at[j], sem))
        for cp in cps:
            cp.wait()
        for j in range(RPT):
            for v in range(NVEC):
                sl = pl.ds(v * 16, 16)
                acc = acc + uvals[j, sl] * mvals[j, sl]
        return acc

    acc = lax.fori_loop(0, E, dim_step, jnp.zeros((16,), jnp.float32))
    accv[...] = acc

    pltpu.sync_copy(accv, shared.at[s])
    plsc.subcore_barrier()
    pltpu.sync_copy(shared, sumv)
    tot = jnp.zeros((16,), jnp.float32)
    for j in range(16):
        tot = tot + sumv[j, :]
    lanes = lax.iota(jnp.int32, 16)
    dot = tot
    for k in (8, 4, 2, 1):
        accv[...] = dot
        dot = dot + plsc.load_gather(accv, [lax.bitwise_and(lanes + k, 15)])

    cps = []
    for t in range(RPW):
        cps.append(pltpu.async_copy(
            ubias.at[uidx_v.at[c * RPW + t]], ubv.at[t], sem))
        cps.append(pltpu.async_copy(
            mbias.at[midx_v.at[c * RPW + t]], mbv.at[t], sem))
    for cp in cps:
        cp.wait()

    for t in range(RPW):
        for v in range(NVEC):
            sl = pl.ds(v * 16, 16)
            x = dot + ubv[t, sl] + mbv[t, sl]
            outv[t, sl] = 1.0 / (1.0 + jnp.exp(-x))
    pltpu.sync_copy(outv, out.at[s, pl.ds(c * RPW, RPW)])


_mesh = plsc.VectorSubcoreMesh(
    core_axis_name="c", subcore_axis_name="s", num_cores=2, num_subcores=16)

_sc_call = functools.partial(
    pl.kernel,
    out_type=jax.ShapeDtypeStruct((16, RPT, CHUNK), jnp.float32),
    mesh=_mesh,
    compiler_params=pltpu.CompilerParams(
        needs_layout_passes=False, use_tc_tiling_on_sc=False),
    scratch_types=[
        pltpu.VMEM((RPT, CHUNK), jnp.int32),
        pltpu.VMEM((RPT, CHUNK), jnp.int32),
        pltpu.VMEM((2, RPT, CHUNK), jnp.int32),
        pltpu.VMEM((RPT, CHUNK), jnp.float32),
        pltpu.VMEM((RPT, CHUNK), jnp.float32),
        pltpu.VMEM((16,), jnp.float32),
        pltpu.VMEM_SHARED((16, 16), jnp.float32),
        pltpu.VMEM((16, 16), jnp.float32),
        pltpu.VMEM((RPW, CHUNK), jnp.float32),
        pltpu.VMEM((RPW, CHUNK), jnp.float32),
        pltpu.VMEM((RPW, CHUNK), jnp.float32),
        pltpu.SemaphoreType.DMA,
    ],
)(_body)


def kernel(inputs, user_embedding, user_bias, movie_embedding, movie_bias):
    idx = inputs.astype(jnp.int32)
    uidx = idx[:, 0].reshape(ROWS, CHUNK)
    midx = idx[:, 1].reshape(ROWS, CHUNK)
    uflat = user_embedding.T.reshape(-1)
    mflat = movie_embedding.T.reshape(-1)
    ub = user_bias.reshape(-1)
    mb = movie_bias.reshape(-1)
    out = _sc_call(uflat, mflat, ub, mb, uidx, midx)
    return out.reshape(B, 1)


# speedup vs baseline: 3.3098x; 3.3098x over previous
import functools

import jax
import jax.numpy as jnp
from jax import lax
from jax.experimental import pallas as pl
from jax.experimental.pallas import tpu as pltpu
from jax.experimental.pallas import tpu_sc as plsc

B = 16384
E = 16
ROWS = 128
CHUNK = 128
RPT = 8
RPW = 4
NVEC = CHUNK // 16


def _body(uemb, ubias, memb, mbias, uidx, midx, out,
          uidx_v, midx_v, urows_v, mrows_v, accv, shared, sumv,
          ubv, mbv, outv, sem):
    c = lax.axis_index("c")
    s = lax.axis_index("s")

    pltpu.sync_copy(uidx.at[pl.ds(s * RPT, RPT)], uidx_v)
    pltpu.sync_copy(midx.at[pl.ds(s * RPT, RPT)], midx_v)

    cps = []
    for j in range(RPT):
        cps.append(pltpu.async_copy(uemb.at[uidx_v.at[j]], urows_v.at[j], sem))
        cps.append(pltpu.async_copy(memb.at[midx_v.at[j]], mrows_v.at[j], sem))
    for cp in cps:
        cp.wait()

    acc = jnp.zeros((16,), jnp.float32)
    for j in range(RPT):
        def dot_step(i, a, j=j):
            return a + urows_v[j, i, :] * mrows_v[j, i, :]
        acc = lax.fori_loop(0, CHUNK, dot_step, acc)
    accv[...] = acc

    pltpu.sync_copy(accv, shared.at[s])
    plsc.subcore_barrier()
    pltpu.sync_copy(shared, sumv)
    tot = jnp.zeros((16,), jnp.float32)
    for j in range(16):
        tot = tot + sumv[j, :]
    lanes = lax.iota(jnp.int32, 16)
    dot = tot
    for k in (8, 4, 2, 1):
        accv[...] = dot
        dot = dot + plsc.load_gather(accv, [lax.bitwise_and(lanes + k, 15)])

    cps = []
    for t in range(RPW):
        cps.append(pltpu.async_copy(ubias.at[uidx_v.at[c * RPW + t]], ubv.at[t], sem))
        cps.append(pltpu.async_copy(mbias.at[midx_v.at[c * RPW + t]], mbv.at[t], sem))
    for cp in cps:
        cp.wait()

    for t in range(RPW):
        for v in range(NVEC):
            sl = pl.ds(v * 16, 16)
            x = dot + ubv[t, sl] + mbv[t, sl]
            outv[t, sl] = 1.0 / (1.0 + jnp.exp(-x))
    pltpu.sync_copy(outv, out.at[pl.ds(s * RPT + c * RPW, RPW)])


_mesh = plsc.VectorSubcoreMesh(
    core_axis_name="c", subcore_axis_name="s", num_cores=2, num_subcores=16)

_sc_call = functools.partial(
    pl.kernel,
    out_type=jax.ShapeDtypeStruct((ROWS, CHUNK), jnp.float32),
    mesh=_mesh,
    compiler_params=pltpu.CompilerParams(
        needs_layout_passes=False, use_tc_tiling_on_sc=False),
    scratch_types=[
        pltpu.VMEM((RPT, CHUNK), jnp.int32),
        pltpu.VMEM((RPT, CHUNK), jnp.int32),
        pltpu.VMEM((RPT, CHUNK, E), jnp.float32),
        pltpu.VMEM((RPT, CHUNK, E), jnp.float32),
        pltpu.VMEM((16,), jnp.float32),
        pltpu.VMEM_SHARED((16, 16), jnp.float32),
        pltpu.VMEM((16, 16), jnp.float32),
        pltpu.VMEM((RPW, CHUNK), jnp.float32),
        pltpu.VMEM((RPW, CHUNK), jnp.float32),
        pltpu.VMEM((RPW, CHUNK), jnp.float32),
        pltpu.SemaphoreType.DMA,
    ],
)(_body)


def kernel(inputs, user_embedding, user_bias, movie_embedding, movie_bias):
    idx = inputs.astype(jnp.int32)
    uidx = idx[:, 0].reshape(ROWS, CHUNK)
    midx = idx[:, 1].reshape(ROWS, CHUNK)
    ub = user_bias.reshape(-1)
    mb = movie_bias.reshape(-1)
    out = _sc_call(user_embedding, ub, movie_embedding, mb, uidx, midx)
    return out.reshape(B, 1)
